# Initial kernel scaffold; baseline (speedup 1.0000x reference)
import jax
import jax.numpy as jnp
from jax.experimental import pallas as pl


def kernel(user_emb, item_emb, edge_weight, edge_index):
    raise NotImplementedError("write your pallas kernel here")


# speedup vs baseline: 7.2986x; 7.2986x over previous
import jax
import jax.numpy as jnp
from jax import lax
from jax.experimental import pallas as pl
from jax.experimental.pallas import tpu as pltpu
from jax.experimental.pallas import tpu_sc as plsc

USER_N = 60000
ITEM_N = 40000
NODES = USER_N + ITEM_N
HALF = NODES // 2
EMB = 32
E = 1600000
NC = 2
NS = 16
CHUNK_ROWS = 4
CHUNK_E = CHUNK_ROWS * 128
CPT = 196
EP = NS * CPT * CHUNK_E
RP = EP // 128
WB = 200
WB_CHUNKS = HALF // WB
WB_ITERS = -(-WB_CHUNKS // NS)

_mesh = plsc.VectorSubcoreMesh(core_axis_name="c", subcore_axis_name="s")


def _layer_body(ego, srcr, dstr, wr, out, acc, src_b, dst_b, w_b, rows,
                zbuf, gsem):
    c = lax.axis_index("c")
    s = lax.axis_index("s")
    base_node = c * HALF

    def z_body(i, carry):
        zbuf[i, pl.ds(0, 16)] = jnp.zeros((16,), jnp.float32)
        zbuf[i, pl.ds(16, 16)] = jnp.zeros((16,), jnp.float32)
        return carry

    lax.fori_loop(0, WB, z_body, None)
    for j in range(WB_ITERS):
        m = j * NS + s

        @pl.when(m < WB_CHUNKS)
        def _():
            pltpu.sync_copy(zbuf, acc.at[pl.ds(m * WB, WB)])

    plsc.subcore_barrier()

    def chunk_body(g, carry):
        r0 = (s * CPT + g) * CHUNK_ROWS
        pltpu.sync_copy(srcr.at[pl.ds(r0, CHUNK_ROWS)], src_b)
        pltpu.sync_copy(dstr.at[pl.ds(r0, CHUNK_ROWS)], dst_b)
        pltpu.sync_copy(wr.at[pl.ds(r0, CHUNK_ROWS)], w_b)

        for i in range(CHUNK_ROWS):
            for k in range(8):
                sl = pl.ds(k * 16, 16)
                d = dst_b[i, sl] - base_node
                m = (d >= 0) & (d < HALF)
                w_b[i, sl] = jnp.where(m, w_b[i, sl], 0.0)
                dst_b[i, sl] = jnp.where(m, d, 0)

        cps = []
        for j in range(CHUNK_ROWS):
            cps.append(pltpu.async_copy(
                ego.at[src_b.at[j]], rows.at[pl.ds(j * 128, 128)], gsem))
        for cp in cps:
            cp.wait()

        def mul_row(i, carry2):
            def mul_grp(k, carry3):
                wv = w_b[i, pl.ds(k * 16, 16)]
                for l in range(16):
                    wsc = wv[l]
                    e = i * 128 + k * 16 + l
                    rows[e, pl.ds(0, 16)] = rows[e, pl.ds(0, 16)] * wsc
                    rows[e, pl.ds(16, 16)] = rows[e, pl.ds(16, 16)] * wsc
                return carry3
            return lax.fori_loop(0, 8, mul_grp, carry2)

        lax.fori_loop(0, CHUNK_ROWS, mul_row, None)

        for j in range(CHUNK_ROWS):
            pltpu.sync_copy(rows.at[pl.ds(j * 128, 128)],
                            acc.at[dst_b.at[j]], add=True)
        return carry

    lax.fori_loop(0, CPT, chunk_body, None)
    plsc.subcore_barrier()

    for j in range(WB_ITERS):
        m = j * NS + s

        @pl.when(m < WB_CHUNKS)
        def _():
            pltpu.sync_copy(acc.at[pl.ds(m * WB, WB)],
                            out.at[pl.ds(base_node + m * WB, WB)])


_layer = pl.kernel(
    _layer_body,
    out_type=jax.ShapeDtypeStruct((NODES, EMB), jnp.float32),
    mesh=_mesh,
    compiler_params=pltpu.CompilerParams(use_tc_tiling_on_sc=False),
    scratch_types=[
        pltpu.VMEM_SHARED((HALF, EMB), jnp.float32),
        pltpu.VMEM((CHUNK_ROWS, 128), jnp.int32),
        pltpu.VMEM((CHUNK_ROWS, 128), jnp.int32),
        pltpu.VMEM((CHUNK_ROWS, 128), jnp.float32),
        pltpu.VMEM((CHUNK_E, EMB), jnp.float32),
        pltpu.VMEM((WB, EMB), jnp.float32),
        pltpu.SemaphoreType.DMA,
    ],
)


def _mean_body(e1, e2, e3, out, b1, b2, b3):
    c = lax.axis_index("c")
    s = lax.axis_index("s")
    wid = s * NC + c
    n_chunks = NODES // WB
    n_iters = -(-n_chunks // (NC * NS))

    for j in range(n_iters):
        m = j * NC * NS + wid

        @pl.when(m < n_chunks)
        def _():
            sl_rows = pl.ds(m * WB, WB)
            pltpu.sync_copy(e1.at[sl_rows], b1)
            pltpu.sync_copy(e2.at[sl_rows], b2)
            pltpu.sync_copy(e3.at[sl_rows], b3)

            def m_body(i, carry):
                for k in range(2):
                    sl = pl.ds(k * 16, 16)
                    b1[i, sl] = (b1[i, sl] + b2[i, sl] + b3[i, sl]) * (1.0 / 3.0)
                return carry

            lax.fori_loop(0, WB, m_body, None)
            pltpu.sync_copy(b1, out.at[sl_rows])


_mean = pl.kernel(
    _mean_body,
    out_type=jax.ShapeDtypeStruct((NODES, EMB), jnp.float32),
    mesh=_mesh,
    scratch_types=[
        pltpu.VMEM((WB, EMB), jnp.float32),
        pltpu.VMEM((WB, EMB), jnp.float32),
        pltpu.VMEM((WB, EMB), jnp.float32),
    ],
)


def kernel(user_emb, item_emb, edge_weight, edge_index):
    ego = jnp.concatenate([user_emb, item_emb], axis=0)
    src = edge_index[0]
    dst = edge_index[1]
    pad = EP - E
    srcr = jnp.pad(src, (0, pad)).reshape(RP, 128)
    dstr = jnp.pad(dst, (0, pad)).reshape(RP, 128)
    wr = jnp.pad(edge_weight, (0, pad)).reshape(RP, 128)
    e1 = _layer(ego, srcr, dstr, wr)
    e2 = _layer(e1, srcr, dstr, wr)
    e3 = _layer(e2, srcr, dstr, wr)
    fin = _mean(e1, e2, e3)
    return fin[:USER_N], fin[USER_N:]


# speedup vs baseline: 16.9130x; 2.3173x over previous
import functools

import jax
import jax.numpy as jnp
from jax import lax
from jax.experimental import pallas as pl
from jax.experimental.pallas import tpu as pltpu
from jax.experimental.pallas import tpu_sc as plsc

USER_N = 60000
ITEM_N = 40000
NODES = USER_N + ITEM_N
HALF = NODES // 2
EMB = 32
E = 1600000
NC = 2
NS = 16

SCAN_ROWS = 8
SCAN_E = SCAN_ROWS * 128
SPT = 98
EP = NS * SPT * SCAN_E
RP = EP // 128
REG = SPT * 1024 + 1024

UE = 128
NBUF = 4

WB = 80
WB_CHUNKS = HALF // WB
WB_ITERS = -(-WB_CHUNKS // NS)

_mesh = plsc.VectorSubcoreMesh(core_axis_name="c", subcore_axis_name="s")


def _partition_body(srcr, dstr, wr, psrc, pdst, pw, counts,
                    src_b, dst_b, w_b, f_src, f_dst, f_w, cbuf):
    c = lax.axis_index("c")
    s = lax.axis_index("s")
    base_node = c * HALF
    tile_idx = c * NS + s
    base_el = tile_idx * REG

    def scan_chunk(g, carry):
        off, flushes = carry
        r0 = (s * SPT + g) * SCAN_ROWS
        pltpu.sync_copy(srcr.at[pl.ds(r0, SCAN_ROWS)], src_b)
        pltpu.sync_copy(dstr.at[pl.ds(r0, SCAN_ROWS)], dst_b)
        pltpu.sync_copy(wr.at[pl.ds(r0, SCAN_ROWS)], w_b)

        for i in range(SCAN_ROWS):
            for k in range(8):
                sl = pl.ds(k * 16, 16)
                d = dst_b[i, sl] - base_node
                m = (d >= 0) & (d < HALF)
                pos = plsc.cumsum(m.astype(jnp.int32))
                idx = pos + (off - 1)
                plsc.store_scatter(f_src, [idx], src_b[i, sl], mask=m)
                plsc.store_scatter(f_dst, [idx], d, mask=m)
                plsc.store_scatter(f_w, [idx], w_b[i, sl], mask=m)
                off = off + pos[15]

        def do_flush(args):
            off2, fl = args
            dst_el = base_el + fl * 1024
            pltpu.sync_copy(f_src.at[pl.ds(0, 1024)],
                            psrc.at[pl.ds(dst_el, 1024)])
            pltpu.sync_copy(f_dst.at[pl.ds(0, 1024)],
                            pdst.at[pl.ds(dst_el, 1024)])
            pltpu.sync_copy(f_w.at[pl.ds(0, 1024)],
                            pw.at[pl.ds(dst_el, 1024)])
            for j in range(64):
                slo = pl.ds(j * 16, 16)
                shi = pl.ds(1024 + j * 16, 16)
                f_src[slo] = f_src[shi]
                f_dst[slo] = f_dst[shi]
                f_w[slo] = f_w[shi]
            return off2 - 1024, fl + 1

        return lax.cond(off >= 1024, do_flush, lambda a: a, (off, flushes))

    off, flushes = lax.fori_loop(0, SPT, scan_chunk, (jnp.int32(0),
                                                      jnp.int32(0)))

    for j in range(8):
        sl = pl.ds(off + j * 16, 16)
        f_src[sl] = jnp.zeros((16,), jnp.int32)
        f_dst[sl] = jnp.zeros((16,), jnp.int32)
        f_w[sl] = jnp.zeros((16,), jnp.float32)
    dst_el = base_el + flushes * 1024
    pltpu.sync_copy(f_src.at[pl.ds(0, 1024)], psrc.at[pl.ds(dst_el, 1024)])
    pltpu.sync_copy(f_dst.at[pl.ds(0, 1024)], pdst.at[pl.ds(dst_el, 1024)])
    pltpu.sync_copy(f_w.at[pl.ds(0, 1024)], pw.at[pl.ds(dst_el, 1024)])

    cnt = flushes * 8 + (off + UE - 1) // UE
    cbuf[...] = jnp.full((16,), cnt, jnp.int32)
    pltpu.sync_copy(cbuf, counts.at[pl.ds(tile_idx * 16, 16)])


_partition = pl.kernel(
    _partition_body,
    out_type=(
        jax.ShapeDtypeStruct((NC * NS * REG,), jnp.int32),
        jax.ShapeDtypeStruct((NC * NS * REG,), jnp.int32),
        jax.ShapeDtypeStruct((NC * NS * REG,), jnp.float32),
        jax.ShapeDtypeStruct((NC * NS * 16,), jnp.int32),
    ),
    mesh=_mesh,
    compiler_params=pltpu.CompilerParams(use_tc_tiling_on_sc=False,
                                         needs_layout_passes=False),
    scratch_types=[
        pltpu.VMEM((SCAN_ROWS, 128), jnp.int32),
        pltpu.VMEM((SCAN_ROWS, 128), jnp.int32),
        pltpu.VMEM((SCAN_ROWS, 128), jnp.float32),
        pltpu.VMEM((2048,), jnp.int32),
        pltpu.VMEM((2048,), jnp.int32),
        pltpu.VMEM((2048,), jnp.float32),
        pltpu.VMEM((16,), jnp.int32),
    ],
)


def _layer_common(ego, psrc, pdst, pw, counts, out, e1, e2, acc,
                  sbuf, didx, wbuf, rows, zbuf, a1, a2,
                  isems, gsems, ssems, wbsem, final):
    c = lax.axis_index("c")
    s = lax.axis_index("s")
    base_node = c * HALF
    tile_idx = c * NS + s
    base_el = tile_idx * REG

    def z_body(i, carry):
        zbuf[i, pl.ds(0, 16)] = jnp.zeros((16,), jnp.float32)
        zbuf[i, pl.ds(16, 16)] = jnp.zeros((16,), jnp.float32)
        return carry

    lax.fori_loop(0, WB, z_body, None)
    pltpu.sync_copy(counts.at[pl.ds(tile_idx * 16, 16)],
                    sbuf[0].at[pl.ds(0, 16)])
    n_units = sbuf[0][pl.ds(0, 16)][0]
    for j in range(WB_ITERS):
        m = j * NS + s

        @pl.when(m < WB_CHUNKS)
        def _():
            pltpu.async_copy(zbuf, acc.at[pl.ds(m * WB, WB)], wbsem)

    for j in range(WB_ITERS):
        m = j * NS + s

        @pl.when(m < WB_CHUNKS)
        def _():
            pltpu.make_async_copy(zbuf, acc.at[pl.ds(m * WB, WB)],
                                  wbsem).wait()

    plsc.subcore_barrier()

    def fire_idx(u, slot):
        el = base_el + u * UE
        pltpu.async_copy(psrc.at[pl.ds(el, UE)], sbuf[slot], isems[slot])
        pltpu.async_copy(pdst.at[pl.ds(el, UE)], didx[slot].at[0],
                         isems[slot])
        pltpu.async_copy(pw.at[pl.ds(el, UE)], wbuf[slot], isems[slot])

    def wait_idx(u, slot):
        el = base_el + u * UE
        pltpu.make_async_copy(psrc.at[pl.ds(el, UE)], sbuf[slot],
                              isems[slot]).wait()
        pltpu.make_async_copy(pdst.at[pl.ds(el, UE)], didx[slot].at[0],
                              isems[slot]).wait()
        pltpu.make_async_copy(pw.at[pl.ds(el, UE)], wbuf[slot],
                              isems[slot]).wait()

    def fire_gather(slot):
        pltpu.async_copy(ego.at[sbuf[slot]], rows[slot], gsems[slot])

    def wait_gather(slot):
        pltpu.make_async_copy(ego.at[sbuf[slot]], rows[slot],
                              gsems[slot]).wait()

    def fire_scatter(slot):
        pltpu.async_copy(rows[slot], acc.at[didx[slot].at[0]], ssems[slot],
                         add=True)

    def wait_scatter(slot):
        pltpu.make_async_copy(rows[slot], acc.at[didx[slot].at[0]],
                              ssems[slot]).wait()

    @pl.when(n_units > 0)
    def _():
        fire_idx(0, 0)

    @pl.when(n_units > 1)
    def _():
        fire_idx(1, 1)

    @pl.when(n_units > 0)
    def _():
        wait_idx(0, 0)
        fire_gather(0)

    def super_body(si, carry):
        t0 = si * NBUF
        for j in range(NBUF):
            t = t0 + j
            sl = j
            sl1 = (j + 1) % NBUF
            sl2 = (j + 2) % NBUF

            @pl.when(t < n_units)
            def _():
                @pl.when(t + 1 < n_units)
                def _():
                    wait_idx(t + 1, sl1)
                    fire_gather(sl1)

                @pl.when(t + 2 < n_units)
                def _():
                    @pl.when(t >= 2)
                    def _():
                        wait_scatter(sl2)
                    fire_idx(t + 2, sl2)

                wait_gather(sl)

                def mul_grp(k, carry2):
                    wv = wbuf[sl][pl.ds(k * 16, 16)]
                    for l in range(16):
                        e = k * 16 + l
                        wsc = wv[l]
                        rows[sl][e, pl.ds(0, 16)] = (
                            rows[sl][e, pl.ds(0, 16)] * wsc)
                        rows[sl][e, pl.ds(16, 16)] = (
                            rows[sl][e, pl.ds(16, 16)] * wsc)
                    return carry2

                lax.fori_loop(0, UE // 16, mul_grp, None)
                fire_scatter(sl)

        return carry

    lax.fori_loop(0, (n_units + NBUF - 1) // NBUF, super_body, None)

    for j in range(NBUF):
        @pl.when(j < n_units)
        def _():
            pltpu.make_async_copy(rows[j], acc.at[didx[j].at[0]],
                                  ssems[j]).wait()

    plsc.subcore_barrier()

    for j in range(WB_ITERS):
        m = j * NS + s

        @pl.when(m < WB_CHUNKS)
        def _():
            if not final:
                pltpu.async_copy(acc.at[pl.ds(m * WB, WB)],
                                 out.at[pl.ds(base_node + m * WB, WB)],
                                 wbsem)
            else:
                row0 = base_node + m * WB
                pltpu.sync_copy(acc.at[pl.ds(m * WB, WB)], zbuf)
                pltpu.sync_copy(e1.at[pl.ds(row0, WB)], a1)
                pltpu.sync_copy(e2.at[pl.ds(row0, WB)], a2)

                def mean_body(i, carry):
                    for k in range(2):
                        sl = pl.ds(k * 16, 16)
                        zbuf[i, sl] = (zbuf[i, sl] + a1[i, sl]
                                       + a2[i, sl]) * (1.0 / 3.0)
                    return carry

                lax.fori_loop(0, WB, mean_body, None)
                pltpu.sync_copy(zbuf, out.at[pl.ds(row0, WB)])

    if not final:
        for j in range(WB_ITERS):
            m = j * NS + s

            @pl.when(m < WB_CHUNKS)
            def _():
                pltpu.make_async_copy(acc.at[pl.ds(m * WB, WB)],
                                      out.at[pl.ds(base_node + m * WB, WB)],
                                      wbsem).wait()


def _layer_body(ego, psrc, pdst, pw, counts, out, acc, sbuf0, sbuf1, sbuf2,
                sbuf3, didx0, didx1, didx2, didx3, wbuf0, wbuf1, wbuf2,
                wbuf3, rows0, rows1, rows2, rows3, zbuf, isem0, isem1,
                isem2, isem3, gsem0, gsem1, gsem2, gsem3, ssem0, ssem1,
                ssem2, ssem3, wbsem):
    _layer_common(ego, psrc, pdst, pw, counts, out, None, None, acc,
                  [sbuf0, sbuf1, sbuf2, sbuf3],
                  [didx0, didx1, didx2, didx3],
                  [wbuf0, wbuf1, wbuf2, wbuf3],
                  [rows0, rows1, rows2, rows3], zbuf, None, None,
                  [isem0, isem1, isem2, isem3],
                  [gsem0, gsem1, gsem2, gsem3],
                  [ssem0, ssem1, ssem2, ssem3], wbsem, final=False)


def _layer_final_body(ego, psrc, pdst, pw, counts, e1, e2, out, acc, sbuf0,
                      sbuf1, sbuf2, sbuf3, didx0, didx1, didx2, didx3,
                      wbuf0, wbuf1, wbuf2, wbuf3, rows0, rows1, rows2,
                      rows3, zbuf, a1, a2, isem0, isem1, isem2, isem3,
                      gsem0, gsem1, gsem2, gsem3, ssem0, ssem1, ssem2,
                      ssem3, wbsem):
    _layer_common(ego, psrc, pdst, pw, counts, out, e1, e2, acc,
                  [sbuf0, sbuf1, sbuf2, sbuf3],
                  [didx0, didx1, didx2, didx3],
                  [wbuf0, wbuf1, wbuf2, wbuf3],
                  [rows0, rows1, rows2, rows3], zbuf, a1, a2,
                  [isem0, isem1, isem2, isem3],
                  [gsem0, gsem1, gsem2, gsem3],
                  [ssem0, ssem1, ssem2, ssem3], wbsem, final=True)


_ring_scratch = (
    [pltpu.VMEM((UE,), jnp.int32) for _ in range(NBUF)]
    + [pltpu.VMEM((1, UE), jnp.int32) for _ in range(NBUF)]
    + [pltpu.VMEM((UE,), jnp.float32) for _ in range(NBUF)]
    + [pltpu.VMEM((UE, EMB), jnp.float32) for _ in range(NBUF)]
)

_sem_scratch = [pltpu.SemaphoreType.DMA] * (3 * NBUF + 1)

_layer = pl.kernel(
    _layer_body,
    out_type=jax.ShapeDtypeStruct((NODES, EMB), jnp.float32),
    mesh=_mesh,
    compiler_params=pltpu.CompilerParams(use_tc_tiling_on_sc=False),
    scratch_types=(
        [pltpu.VMEM_SHARED((HALF, EMB), jnp.float32)]
        + _ring_scratch
        + [pltpu.VMEM((WB, EMB), jnp.float32)]
        + _sem_scratch
    ),
)

_layer_final = pl.kernel(
    _layer_final_body,
    out_type=jax.ShapeDtypeStruct((NODES, EMB), jnp.float32),
    mesh=_mesh,
    compiler_params=pltpu.CompilerParams(use_tc_tiling_on_sc=False),
    scratch_types=(
        [pltpu.VMEM_SHARED((HALF, EMB), jnp.float32)]
        + _ring_scratch
        + [pltpu.VMEM((WB, EMB), jnp.float32),
           pltpu.VMEM((WB, EMB), jnp.float32),
           pltpu.VMEM((WB, EMB), jnp.float32)]
        + _sem_scratch
    ),
)


def kernel(user_emb, item_emb, edge_weight, edge_index):
    ego = jnp.concatenate([user_emb, item_emb], axis=0)
    src = edge_index[0]
    dst = edge_index[1]
    pad = EP - E
    srcr = jnp.pad(src, (0, pad)).reshape(RP, 128)
    dstr = jnp.pad(dst, (0, pad)).reshape(RP, 128)
    wr = jnp.pad(edge_weight, (0, pad)).reshape(RP, 128)
    psrc, pdst, pw, counts = _partition(srcr, dstr, wr)
    e1 = _layer(ego, psrc, pdst, pw, counts)
    e2 = _layer(e1, psrc, pdst, pw, counts)
    fin = _layer_final(e2, psrc, pdst, pw, counts, e1, e2)
    return fin[:USER_N], fin[USER_N:]


# speedup vs baseline: 19.0724x; 1.1277x over previous
import functools

import jax
import jax.numpy as jnp
from jax import lax
from jax.experimental import pallas as pl
from jax.experimental.pallas import tpu as pltpu
from jax.experimental.pallas import tpu_sc as plsc

USER_N = 60000
ITEM_N = 40000
NODES = USER_N + ITEM_N
HALF = NODES // 2
EMB = 32
E = 1600000
NC = 2
NS = 16

SCAN_ROWS = 8
SCAN_E = SCAN_ROWS * 128
SPT = 98
EP = NS * SPT * SCAN_E
RP = EP // 128
REG = SPT * 1024 + 1024

UE = 128
NBUF = 4

WB = 80
WB_CHUNKS = HALF // WB
WB_ITERS = -(-WB_CHUNKS // NS)

WF = 40
WF_CHUNKS = HALF // WF
WF_ITERS = -(-WF_CHUNKS // NS)

_mesh = plsc.VectorSubcoreMesh(core_axis_name="c", subcore_axis_name="s")


FRING = 4096


def _partition_body(srcr, dstr, wr, psrc, pdst, pw, counts,
                    src_b0, dst_b0, w_b0, src_b1, dst_b1, w_b1,
                    f_src, f_dst, f_w, cbuf,
                    in_sem0, in_sem1, fl_sem0, fl_sem1):
    c = lax.axis_index("c")
    s = lax.axis_index("s")
    base_node = c * HALF
    tile_idx = c * NS + s
    base_el = tile_idx * REG

    src_b = [src_b0, src_b1]
    dst_b = [dst_b0, dst_b1]
    w_b = [w_b0, w_b1]
    in_sems = [in_sem0, in_sem1]
    fl_sems = [fl_sem0, fl_sem1]

    def fire_in(g, slot):
        r0 = (s * SPT + g) * SCAN_ROWS
        pltpu.async_copy(srcr.at[pl.ds(r0, SCAN_ROWS)], src_b[slot],
                         in_sems[slot])
        pltpu.async_copy(dstr.at[pl.ds(r0, SCAN_ROWS)], dst_b[slot],
                         in_sems[slot])
        pltpu.async_copy(wr.at[pl.ds(r0, SCAN_ROWS)], w_b[slot],
                         in_sems[slot])

    def wait_in(g, slot):
        r0 = (s * SPT + g) * SCAN_ROWS
        pltpu.make_async_copy(srcr.at[pl.ds(r0, SCAN_ROWS)], src_b[slot],
                              in_sems[slot]).wait()
        pltpu.make_async_copy(dstr.at[pl.ds(r0, SCAN_ROWS)], dst_b[slot],
                              in_sems[slot]).wait()
        pltpu.make_async_copy(wr.at[pl.ds(r0, SCAN_ROWS)], w_b[slot],
                              in_sems[slot]).wait()

    def wait_flush(sem):
        pltpu.make_async_copy(f_src.at[pl.ds(0, 1024)],
                              psrc.at[pl.ds(base_el, 1024)], sem).wait()
        pltpu.make_async_copy(f_dst.at[pl.ds(0, 1024)],
                              pdst.at[pl.ds(base_el, 1024)], sem).wait()
        pltpu.make_async_copy(f_w.at[pl.ds(0, 1024)],
                              pw.at[pl.ds(base_el, 1024)], sem).wait()

    def fire_flush(fq, sem):
        q0 = (fq & 3) * 1024
        dst_el = base_el + fq * 1024
        pltpu.async_copy(f_src.at[pl.ds(q0, 1024)],
                         psrc.at[pl.ds(dst_el, 1024)], sem)
        pltpu.async_copy(f_dst.at[pl.ds(q0, 1024)],
                         pdst.at[pl.ds(dst_el, 1024)], sem)
        pltpu.async_copy(f_w.at[pl.ds(q0, 1024)],
                         pw.at[pl.ds(dst_el, 1024)], sem)

    fire_in(0, 0)
    fire_in(1, 1)

    def sub_chunk(g, slot, carry):
        tot, fq = carry
        wait_in(g, slot)
        for i in range(SCAN_ROWS):
            for k in range(8):
                sl = pl.ds(k * 16, 16)
                d = dst_b[slot][i, sl] - base_node
                m = (d >= 0) & (d < HALF)
                pos = plsc.cumsum(m.astype(jnp.int32))
                idx = (pos + (tot - 1)) & (FRING - 1)
                plsc.store_scatter(f_src, [idx], src_b[slot][i, sl], mask=m)
                plsc.store_scatter(f_dst, [idx], d, mask=m)
                plsc.store_scatter(f_w, [idx], w_b[slot][i, sl], mask=m)
                tot = tot + pos[15]

        @pl.when(g + 2 < SPT)
        def _():
            fire_in(g + 2, slot)

        unflushed = tot - fq * 1024
        need = unflushed >= 1024
        even = (fq & 1) == 0

        @pl.when(need & even & (fq >= 2))
        def _():
            wait_flush(fl_sem0)

        @pl.when(need & even)
        def _():
            fire_flush(fq, fl_sem0)

        @pl.when(need & (~even) & (fq >= 2))
        def _():
            wait_flush(fl_sem1)

        @pl.when(need & (~even))
        def _():
            fire_flush(fq, fl_sem1)

        return tot, fq + (unflushed >> 10)

    def super_chunk(k, carry):
        carry = sub_chunk(2 * k, 0, carry)
        carry = sub_chunk(2 * k + 1, 1, carry)
        return carry

    tot, fq = lax.fori_loop(0, SPT // 2, super_chunk,
                            (jnp.int32(0), jnp.int32(0)))

    evenq = (fq & 1) == 0

    @pl.when((fq >= 1) & evenq)
    def _():
        wait_flush(fl_sem1)

    @pl.when((fq >= 1) & (~evenq))
    def _():
        wait_flush(fl_sem0)

    @pl.when((fq >= 2) & evenq)
    def _():
        wait_flush(fl_sem0)

    @pl.when((fq >= 2) & (~evenq))
    def _():
        wait_flush(fl_sem1)

    lanes = lax.iota(jnp.int32, 16)
    mall = lanes >= 0
    zi = jnp.zeros((16,), jnp.int32)
    zf = jnp.zeros((16,), jnp.float32)
    for j in range(8):
        idxv = (lanes + (tot + j * 16)) & (FRING - 1)
        plsc.store_scatter(f_src, [idxv], zi, mask=mall)
        plsc.store_scatter(f_dst, [idxv], zi, mask=mall)
        plsc.store_scatter(f_w, [idxv], zf, mask=mall)

    q0 = (fq & 3) * 1024
    dst_el = base_el + fq * 1024
    pltpu.sync_copy(f_src.at[pl.ds(q0, 1024)], psrc.at[pl.ds(dst_el, 1024)])
    pltpu.sync_copy(f_dst.at[pl.ds(q0, 1024)], pdst.at[pl.ds(dst_el, 1024)])
    pltpu.sync_copy(f_w.at[pl.ds(q0, 1024)], pw.at[pl.ds(dst_el, 1024)])

    rem = tot - fq * 1024
    cnt = fq * 8 + ((rem + UE - 1) >> 7)
    cbuf[...] = jnp.full((16,), cnt, jnp.int32)
    pltpu.sync_copy(cbuf, counts.at[pl.ds(tile_idx * 16, 16)])


_partition = pl.kernel(
    _partition_body,
    out_type=(
        jax.ShapeDtypeStruct((NC * NS * REG,), jnp.int32),
        jax.ShapeDtypeStruct((NC * NS * REG,), jnp.int32),
        jax.ShapeDtypeStruct((NC * NS * REG,), jnp.float32),
        jax.ShapeDtypeStruct((NC * NS * 16,), jnp.int32),
    ),
    mesh=_mesh,
    compiler_params=pltpu.CompilerParams(use_tc_tiling_on_sc=False,
                                         needs_layout_passes=False),
    scratch_types=(
        [pltpu.VMEM((SCAN_ROWS, 128), jnp.int32),
         pltpu.VMEM((SCAN_ROWS, 128), jnp.int32),
         pltpu.VMEM((SCAN_ROWS, 128), jnp.float32)] * 2
        + [pltpu.VMEM((FRING,), jnp.int32),
           pltpu.VMEM((FRING,), jnp.int32),
           pltpu.VMEM((FRING,), jnp.float32),
           pltpu.VMEM((16,), jnp.int32)]
        + [pltpu.SemaphoreType.DMA] * 4
    ),
)


def _layer_common(ego, psrc, pdst, pw, counts, out, e1, e2, acc,
                  sbuf, didx, wbuf, rows, zbuf, wsets,
                  isems, gsems, ssems, wbsem, ldsems, stsems, final):
    c = lax.axis_index("c")
    s = lax.axis_index("s")
    base_node = c * HALF
    tile_idx = c * NS + s
    base_el = tile_idx * REG

    zw = WF if final else WB
    zn = WF_ITERS if final else WB_ITERS
    zc = WF_CHUNKS if final else WB_CHUNKS

    def z_body(i, carry):
        zbuf[i, pl.ds(0, 16)] = jnp.zeros((16,), jnp.float32)
        zbuf[i, pl.ds(16, 16)] = jnp.zeros((16,), jnp.float32)
        return carry

    lax.fori_loop(0, zw, z_body, None)
    pltpu.sync_copy(counts.at[pl.ds(tile_idx * 16, 16)],
                    sbuf[0].at[pl.ds(0, 16)])
    n_units = sbuf[0][pl.ds(0, 16)][0]
    for j in range(zn):
        m = j * NS + s

        @pl.when(m < zc)
        def _():
            pltpu.async_copy(zbuf, acc.at[pl.ds(m * zw, zw)], wbsem)

    for j in range(zn):
        m = j * NS + s

        @pl.when(m < zc)
        def _():
            pltpu.make_async_copy(zbuf, acc.at[pl.ds(m * zw, zw)],
                                  wbsem).wait()

    plsc.subcore_barrier()

    def fire_idx(u, slot):
        el = base_el + u * UE
        pltpu.async_copy(psrc.at[pl.ds(el, UE)], sbuf[slot], isems[slot])
        pltpu.async_copy(pdst.at[pl.ds(el, UE)], didx[slot].at[0],
                         isems[slot])
        pltpu.async_copy(pw.at[pl.ds(el, UE)], wbuf[slot], isems[slot])

    def wait_idx(u, slot):
        el = base_el + u * UE
        pltpu.make_async_copy(psrc.at[pl.ds(el, UE)], sbuf[slot],
                              isems[slot]).wait()
        pltpu.make_async_copy(pdst.at[pl.ds(el, UE)], didx[slot].at[0],
                              isems[slot]).wait()
        pltpu.make_async_copy(pw.at[pl.ds(el, UE)], wbuf[slot],
                              isems[slot]).wait()

    def fire_gather(slot):
        pltpu.async_copy(ego.at[sbuf[slot]], rows[slot], gsems[slot])

    def wait_gather(slot):
        pltpu.make_async_copy(ego.at[sbuf[slot]], rows[slot],
                              gsems[slot]).wait()

    def fire_scatter(slot):
        pltpu.async_copy(rows[slot], acc.at[didx[slot].at[0]], ssems[slot],
                         add=True)

    def wait_scatter(slot):
        pltpu.make_async_copy(rows[slot], acc.at[didx[slot].at[0]],
                              ssems[slot]).wait()

    @pl.when(n_units > 0)
    def _():
        fire_idx(0, 0)

    @pl.when(n_units > 1)
    def _():
        fire_idx(1, 1)

    @pl.when(n_units > 0)
    def _():
        wait_idx(0, 0)
        fire_gather(0)

    def super_body(si, carry):
        t0 = si * NBUF
        for j in range(NBUF):
            t = t0 + j
            sl = j
            sl1 = (j + 1) % NBUF
            sl2 = (j + 2) % NBUF

            @pl.when(t < n_units)
            def _():
                @pl.when(t + 1 < n_units)
                def _():
                    wait_idx(t + 1, sl1)
                    fire_gather(sl1)

                @pl.when(t + 2 < n_units)
                def _():
                    @pl.when(t >= 2)
                    def _():
                        wait_scatter(sl2)
                    fire_idx(t + 2, sl2)

                wait_gather(sl)

                def mul_grp(k, carry2):
                    wv = wbuf[sl][pl.ds(k * 16, 16)]
                    for l in range(16):
                        e = k * 16 + l
                        wsc = wv[l]
                        rows[sl][e, pl.ds(0, 16)] = (
                            rows[sl][e, pl.ds(0, 16)] * wsc)
                        rows[sl][e, pl.ds(16, 16)] = (
                            rows[sl][e, pl.ds(16, 16)] * wsc)
                    return carry2

                lax.fori_loop(0, UE // 16, mul_grp, None)
                fire_scatter(sl)

        return carry

    lax.fori_loop(0, (n_units + NBUF - 1) // NBUF, super_body, None)

    for j in range(NBUF):
        @pl.when(j < n_units)
        def _():
            pltpu.make_async_copy(rows[j], acc.at[didx[j].at[0]],
                                  ssems[j]).wait()

    plsc.subcore_barrier()

    if not final:
        for j in range(WB_ITERS):
            m = j * NS + s

            @pl.when(m < WB_CHUNKS)
            def _():
                pltpu.async_copy(acc.at[pl.ds(m * WB, WB)],
                                 out.at[pl.ds(base_node + m * WB, WB)],
                                 wbsem)

        for j in range(WB_ITERS):
            m = j * NS + s

            @pl.when(m < WB_CHUNKS)
            def _():
                pltpu.make_async_copy(acc.at[pl.ds(m * WB, WB)],
                                      out.at[pl.ds(base_node + m * WB, WB)],
                                      wbsem).wait()
    else:
        def valid(j):
            return (j * NS + s) < WF_CHUNKS

        def f_loads(j, t):
            _, xb, yb = wsets[t]
            row0 = base_node + (j * NS + s) * WF
            pltpu.async_copy(e1.at[pl.ds(row0, WF)], xb, ldsems[t])
            pltpu.async_copy(e2.at[pl.ds(row0, WF)], yb, ldsems[t])

        def w_loads(j, t):
            _, xb, yb = wsets[t]
            row0 = base_node + (j * NS + s) * WF
            pltpu.make_async_copy(e1.at[pl.ds(row0, WF)], xb,
                                  ldsems[t]).wait()
            pltpu.make_async_copy(e2.at[pl.ds(row0, WF)], yb,
                                  ldsems[t]).wait()

        def f_store(j, t):
            ab = wsets[t][0]
            row0 = base_node + (j * NS + s) * WF
            pltpu.async_copy(ab, out.at[pl.ds(row0, WF)], stsems[t])

        def w_store(j, t):
            ab = wsets[t][0]
            row0 = base_node + (j * NS + s) * WF
            pltpu.make_async_copy(ab, out.at[pl.ds(row0, WF)],
                                  stsems[t]).wait()

        @pl.when(valid(0))
        def _():
            f_loads(0, 0)

        for j in range(WF_ITERS):
            t = j & 1
            if j >= 1:
                @pl.when(valid(j - 1))
                def _():
                    w_store(j - 1, 1 - t)

            if j + 1 < WF_ITERS:
                @pl.when(valid(j + 1))
                def _():
                    f_loads(j + 1, 1 - t)

            @pl.when(valid(j))
            def _():
                ab, xb, yb = wsets[t]
                pltpu.sync_copy(acc.at[pl.ds((j * NS + s) * WF, WF)], ab)
                w_loads(j, t)

                def mean_body(i, carry):
                    for k in range(2):
                        sl = pl.ds(k * 16, 16)
                        ab[i, sl] = (ab[i, sl] + xb[i, sl]
                                     + yb[i, sl]) * (1.0 / 3.0)
                    return carry

                lax.fori_loop(0, WF, mean_body, None)
                f_store(j, t)

        @pl.when(valid(WF_ITERS - 1))
        def _():
            w_store(WF_ITERS - 1, (WF_ITERS - 1) & 1)


def _layer_body(ego, psrc, pdst, pw, counts, out, acc, sbuf0, sbuf1, sbuf2,
                sbuf3, didx0, didx1, didx2, didx3, wbuf0, wbuf1, wbuf2,
                wbuf3, rows0, rows1, rows2, rows3, zbuf, isem0, isem1,
                isem2, isem3, gsem0, gsem1, gsem2, gsem3, ssem0, ssem1,
                ssem2, ssem3, wbsem):
    _layer_common(ego, psrc, pdst, pw, counts, out, None, None, acc,
                  [sbuf0, sbuf1, sbuf2, sbuf3],
                  [didx0, didx1, didx2, didx3],
                  [wbuf0, wbuf1, wbuf2, wbuf3],
                  [rows0, rows1, rows2, rows3], zbuf, None,
                  [isem0, isem1, isem2, isem3],
                  [gsem0, gsem1, gsem2, gsem3],
                  [ssem0, ssem1, ssem2, ssem3], wbsem, None, None,
                  final=False)


def _layer_final_body(ego, psrc, pdst, pw, counts, e1, e2, out, acc, sbuf0,
                      sbuf1, sbuf2, sbuf3, didx0, didx1, didx2, didx3,
                      wbuf0, wbuf1, wbuf2, wbuf3, rows0, rows1, rows2,
                      rows3, zbuf, ab0, xb0, yb0, ab1, xb1, yb1,
                      isem0, isem1, isem2, isem3,
                      gsem0, gsem1, gsem2, gsem3, ssem0, ssem1, ssem2,
                      ssem3, wbsem):
    _layer_common(ego, psrc, pdst, pw, counts, out, e1, e2, acc,
                  [sbuf0, sbuf1, sbuf2, sbuf3],
                  [didx0, didx1, didx2, didx3],
                  [wbuf0, wbuf1, wbuf2, wbuf3],
                  [rows0, rows1, rows2, rows3], zbuf,
                  [(ab0, xb0, yb0), (ab1, xb1, yb1)],
                  [isem0, isem1, isem2, isem3],
                  [gsem0, gsem1, gsem2, gsem3],
                  [ssem0, ssem1, ssem2, ssem3], wbsem,
                  [isem0, isem1], [isem2, isem3], final=True)


_ring_scratch = (
    [pltpu.VMEM((UE,), jnp.int32) for _ in range(NBUF)]
    + [pltpu.VMEM((1, UE), jnp.int32) for _ in range(NBUF)]
    + [pltpu.VMEM((UE,), jnp.float32) for _ in range(NBUF)]
    + [pltpu.VMEM((UE, EMB), jnp.float32) for _ in range(NBUF)]
)

_sem_scratch = [pltpu.SemaphoreType.DMA] * (3 * NBUF + 1)

_layer = pl.kernel(
    _layer_body,
    out_type=jax.ShapeDtypeStruct((NODES, EMB), jnp.float32),
    mesh=_mesh,
    compiler_params=pltpu.CompilerParams(use_tc_tiling_on_sc=False),
    scratch_types=(
        [pltpu.VMEM_SHARED((HALF, EMB), jnp.float32)]
        + _ring_scratch
        + [pltpu.VMEM((WB, EMB), jnp.float32)]
        + _sem_scratch
    ),
)

_layer_final = pl.kernel(
    _layer_final_body,
    out_type=jax.ShapeDtypeStruct((NODES, EMB), jnp.float32),
    mesh=_mesh,
    compiler_params=pltpu.CompilerParams(use_tc_tiling_on_sc=False),
    scratch_types=(
        [pltpu.VMEM_SHARED((HALF, EMB), jnp.float32)]
        + _ring_scratch
        + [pltpu.VMEM((WF, EMB), jnp.float32) for _ in range(7)]
        + _sem_scratch
    ),
)


def kernel(user_emb, item_emb, edge_weight, edge_index):
    ego = jnp.concatenate([user_emb, item_emb], axis=0)
    src = edge_index[0]
    dst = edge_index[1]
    pad = EP - E
    srcr = jnp.pad(src, (0, pad)).reshape(RP, 128)
    dstr = jnp.pad(dst, (0, pad)).reshape(RP, 128)
    wr = jnp.pad(edge_weight, (0, pad)).reshape(RP, 128)
    psrc, pdst, pw, counts = _partition(srcr, dstr, wr)
    e1 = _layer(ego, psrc, pdst, pw, counts)
    e2 = _layer(e1, psrc, pdst, pw, counts)
    fin = _layer_final(e2, psrc, pdst, pw, counts, e1, e2)
    return fin[:USER_N], fin[USER_N:]


# speedup vs baseline: 19.0789x; 1.0003x over previous
import functools

import jax
import jax.numpy as jnp
from jax import lax
from jax.experimental import pallas as pl
from jax.experimental.pallas import tpu as pltpu
from jax.experimental.pallas import tpu_sc as plsc

USER_N = 60000
ITEM_N = 40000
NODES = USER_N + ITEM_N
HALF = NODES // 2
EMB = 32
E = 1600000
NC = 2
NS = 16

SCAN_ROWS = 8
SCAN_E = SCAN_ROWS * 128
SPT = 98
EP = NS * SPT * SCAN_E
RP = EP // 128
REG = SPT * 1024 + 1024

UE = 128
NBUF = 4

WB = 80
WB_CHUNKS = HALF // WB
WB_ITERS = -(-WB_CHUNKS // NS)

WF = 40
WF_CHUNKS = HALF // WF
WF_ITERS = -(-WF_CHUNKS // NS)

_mesh = plsc.VectorSubcoreMesh(core_axis_name="c", subcore_axis_name="s")


FRING = 4096


def _partition_body(srcr, dstr, wr, psrc, pdst, pw, counts,
                    src_b0, dst_b0, w_b0, src_b1, dst_b1, w_b1,
                    f_src, f_dst, f_w, cbuf,
                    in_sem0, in_sem1, fl_sem0, fl_sem1):
    c = lax.axis_index("c")
    s = lax.axis_index("s")
    base_node = c * HALF
    tile_idx = c * NS + s
    base_el = tile_idx * REG

    src_b = [src_b0, src_b1]
    dst_b = [dst_b0, dst_b1]
    w_b = [w_b0, w_b1]
    in_sems = [in_sem0, in_sem1]
    fl_sems = [fl_sem0, fl_sem1]

    def fire_in(g, slot):
        r0 = (s * SPT + g) * SCAN_ROWS
        pltpu.async_copy(srcr.at[pl.ds(r0, SCAN_ROWS)], src_b[slot],
                         in_sems[slot])
        pltpu.async_copy(dstr.at[pl.ds(r0, SCAN_ROWS)], dst_b[slot],
                         in_sems[slot])
        pltpu.async_copy(wr.at[pl.ds(r0, SCAN_ROWS)], w_b[slot],
                         in_sems[slot])

    def wait_in(g, slot):
        r0 = (s * SPT + g) * SCAN_ROWS
        pltpu.make_async_copy(srcr.at[pl.ds(r0, SCAN_ROWS)], src_b[slot],
                              in_sems[slot]).wait()
        pltpu.make_async_copy(dstr.at[pl.ds(r0, SCAN_ROWS)], dst_b[slot],
                              in_sems[slot]).wait()
        pltpu.make_async_copy(wr.at[pl.ds(r0, SCAN_ROWS)], w_b[slot],
                              in_sems[slot]).wait()

    def wait_flush(sem):
        pltpu.make_async_copy(f_src.at[pl.ds(0, 1024)],
                              psrc.at[pl.ds(base_el, 1024)], sem).wait()
        pltpu.make_async_copy(f_dst.at[pl.ds(0, 1024)],
                              pdst.at[pl.ds(base_el, 1024)], sem).wait()
        pltpu.make_async_copy(f_w.at[pl.ds(0, 1024)],
                              pw.at[pl.ds(base_el, 1024)], sem).wait()

    def fire_flush(fq, sem):
        q0 = (fq & 3) * 1024
        dst_el = base_el + fq * 1024
        pltpu.async_copy(f_src.at[pl.ds(q0, 1024)],
                         psrc.at[pl.ds(dst_el, 1024)], sem)
        pltpu.async_copy(f_dst.at[pl.ds(q0, 1024)],
                         pdst.at[pl.ds(dst_el, 1024)], sem)
        pltpu.async_copy(f_w.at[pl.ds(q0, 1024)],
                         pw.at[pl.ds(dst_el, 1024)], sem)

    fire_in(0, 0)
    fire_in(1, 1)

    def sub_chunk(g, slot, carry):
        tot, fq = carry
        wait_in(g, slot)
        for i in range(SCAN_ROWS):
            for k in range(8):
                sl = pl.ds(k * 16, 16)
                d = dst_b[slot][i, sl] - base_node
                m = (d >= 0) & (d < HALF)
                pos = plsc.cumsum(m.astype(jnp.int32))
                idx = (pos + (tot - 1)) & (FRING - 1)
                plsc.store_scatter(f_src, [idx], src_b[slot][i, sl], mask=m)
                plsc.store_scatter(f_dst, [idx], d, mask=m)
                plsc.store_scatter(f_w, [idx], w_b[slot][i, sl], mask=m)
                tot = tot + pos[15]

        @pl.when(g + 2 < SPT)
        def _():
            fire_in(g + 2, slot)

        unflushed = tot - fq * 1024
        need = unflushed >= 1024
        even = (fq & 1) == 0

        @pl.when(need & even & (fq >= 2))
        def _():
            wait_flush(fl_sem0)

        @pl.when(need & even)
        def _():
            fire_flush(fq, fl_sem0)

        @pl.when(need & (~even) & (fq >= 2))
        def _():
            wait_flush(fl_sem1)

        @pl.when(need & (~even))
        def _():
            fire_flush(fq, fl_sem1)

        return tot, fq + (unflushed >> 10)

    def super_chunk(k, carry):
        carry = sub_chunk(2 * k, 0, carry)
        carry = sub_chunk(2 * k + 1, 1, carry)
        return carry

    tot, fq = lax.fori_loop(0, SPT // 2, super_chunk,
                            (jnp.int32(0), jnp.int32(0)))

    evenq = (fq & 1) == 0

    @pl.when((fq >= 1) & evenq)
    def _():
        wait_flush(fl_sem1)

    @pl.when((fq >= 1) & (~evenq))
    def _():
        wait_flush(fl_sem0)

    @pl.when((fq >= 2) & evenq)
    def _():
        wait_flush(fl_sem0)

    @pl.when((fq >= 2) & (~evenq))
    def _():
        wait_flush(fl_sem1)

    lanes = lax.iota(jnp.int32, 16)
    mall = lanes >= 0
    zi = jnp.zeros((16,), jnp.int32)
    zf = jnp.zeros((16,), jnp.float32)
    for j in range(8):
        idxv = (lanes + (tot + j * 16)) & (FRING - 1)
        plsc.store_scatter(f_src, [idxv], zi, mask=mall)
        plsc.store_scatter(f_dst, [idxv], zi, mask=mall)
        plsc.store_scatter(f_w, [idxv], zf, mask=mall)

    q0 = (fq & 3) * 1024
    dst_el = base_el + fq * 1024
    pltpu.sync_copy(f_src.at[pl.ds(q0, 1024)], psrc.at[pl.ds(dst_el, 1024)])
    pltpu.sync_copy(f_dst.at[pl.ds(q0, 1024)], pdst.at[pl.ds(dst_el, 1024)])
    pltpu.sync_copy(f_w.at[pl.ds(q0, 1024)], pw.at[pl.ds(dst_el, 1024)])

    rem = tot - fq * 1024
    cnt = fq * 8 + ((rem + UE - 1) >> 7)
    cbuf[...] = jnp.full((16,), cnt, jnp.int32)
    pltpu.sync_copy(cbuf, counts.at[pl.ds(tile_idx * 16, 16)])


_partition = pl.kernel(
    _partition_body,
    out_type=(
        jax.ShapeDtypeStruct((NC * NS * REG,), jnp.int32),
        jax.ShapeDtypeStruct((NC * NS * REG,), jnp.int32),
        jax.ShapeDtypeStruct((NC * NS * REG,), jnp.float32),
        jax.ShapeDtypeStruct((NC * NS * 16,), jnp.int32),
    ),
    mesh=_mesh,
    compiler_params=pltpu.CompilerParams(use_tc_tiling_on_sc=False,
                                         needs_layout_passes=False),
    scratch_types=(
        [pltpu.VMEM((SCAN_ROWS, 128), jnp.int32),
         pltpu.VMEM((SCAN_ROWS, 128), jnp.int32),
         pltpu.VMEM((SCAN_ROWS, 128), jnp.float32)] * 2
        + [pltpu.VMEM((FRING,), jnp.int32),
           pltpu.VMEM((FRING,), jnp.int32),
           pltpu.VMEM((FRING,), jnp.float32),
           pltpu.VMEM((16,), jnp.int32)]
        + [pltpu.SemaphoreType.DMA] * 4
    ),
)


def _layer_common(ego, psrc, pdst, pw, counts, out, e1, e2, acc,
                  sbuf, didx, wbuf, rows, zbuf, wsets,
                  isems, gsems, ssems, wbsem, ldsems, stsems, final):
    c = lax.axis_index("c")
    s = lax.axis_index("s")
    base_node = c * HALF
    tile_idx = c * NS + s
    base_el = tile_idx * REG

    zw = WF if final else WB
    zn = WF_ITERS if final else WB_ITERS
    zc = WF_CHUNKS if final else WB_CHUNKS

    def z_body(i, carry):
        zbuf[i, pl.ds(0, 16)] = jnp.zeros((16,), jnp.float32)
        zbuf[i, pl.ds(16, 16)] = jnp.zeros((16,), jnp.float32)
        return carry

    lax.fori_loop(0, zw, z_body, None)
    pltpu.sync_copy(counts.at[pl.ds(tile_idx * 16, 16)],
                    sbuf[0].at[pl.ds(0, 16)])
    n_units = sbuf[0][pl.ds(0, 16)][0]
    for j in range(zn):
        m = j * NS + s

        @pl.when(m < zc)
        def _():
            pltpu.async_copy(zbuf, acc.at[pl.ds(m * zw, zw)], wbsem)

    for j in range(zn):
        m = j * NS + s

        @pl.when(m < zc)
        def _():
            pltpu.make_async_copy(zbuf, acc.at[pl.ds(m * zw, zw)],
                                  wbsem).wait()

    plsc.subcore_barrier()

    def fire_idx(u, slot):
        el = base_el + u * UE
        pltpu.async_copy(psrc.at[pl.ds(el, UE)], sbuf[slot], isems[slot])
        pltpu.async_copy(pdst.at[pl.ds(el, UE)], didx[slot].at[0],
                         isems[slot])
        pltpu.async_copy(pw.at[pl.ds(el, UE)], wbuf[slot], isems[slot])

    def wait_idx(u, slot):
        el = base_el + u * UE
        pltpu.make_async_copy(psrc.at[pl.ds(el, UE)], sbuf[slot],
                              isems[slot]).wait()
        pltpu.make_async_copy(pdst.at[pl.ds(el, UE)], didx[slot].at[0],
                              isems[slot]).wait()
        pltpu.make_async_copy(pw.at[pl.ds(el, UE)], wbuf[slot],
                              isems[slot]).wait()

    def fire_gather(slot):
        pltpu.async_copy(ego.at[sbuf[slot]], rows[slot], gsems[slot])

    def wait_gather(slot):
        pltpu.make_async_copy(ego.at[sbuf[slot]], rows[slot],
                              gsems[slot]).wait()

    def fire_scatter(slot):
        pltpu.async_copy(rows[slot], acc.at[didx[slot].at[0]], ssems[slot],
                         add=True)

    def wait_scatter(slot):
        pltpu.make_async_copy(rows[slot], acc.at[didx[slot].at[0]],
                              ssems[slot]).wait()

    @pl.when(n_units > 0)
    def _():
        fire_idx(0, 0)

    @pl.when(n_units > 1)
    def _():
        fire_idx(1, 1)

    @pl.when(n_units > 0)
    def _():
        wait_idx(0, 0)
        fire_gather(0)

    def super_body(si, carry):
        t0 = si * NBUF
        for j in range(NBUF):
            t = t0 + j
            sl = j
            sl1 = (j + 1) % NBUF
            sl2 = (j + 2) % NBUF

            @pl.when(t < n_units)
            def _():
                @pl.when(t + 1 < n_units)
                def _():
                    wait_idx(t + 1, sl1)
                    fire_gather(sl1)

                @pl.when(t + 2 < n_units)
                def _():
                    @pl.when(t >= 2)
                    def _():
                        wait_scatter(sl2)
                    fire_idx(t + 2, sl2)

                wait_gather(sl)

                for k in range(UE // 16):
                    wv = wbuf[sl][pl.ds(k * 16, 16)]
                    for l in range(16):
                        e = k * 16 + l
                        wsc = wv[l]
                        rows[sl][e, pl.ds(0, 16)] = (
                            rows[sl][e, pl.ds(0, 16)] * wsc)
                        rows[sl][e, pl.ds(16, 16)] = (
                            rows[sl][e, pl.ds(16, 16)] * wsc)
                fire_scatter(sl)

        return carry

    lax.fori_loop(0, (n_units + NBUF - 1) // NBUF, super_body, None)

    for j in range(NBUF):
        @pl.when(j < n_units)
        def _():
            pltpu.make_async_copy(rows[j], acc.at[didx[j].at[0]],
                                  ssems[j]).wait()

    plsc.subcore_barrier()

    if not final:
        for j in range(WB_ITERS):
            m = j * NS + s

            @pl.when(m < WB_CHUNKS)
            def _():
                pltpu.async_copy(acc.at[pl.ds(m * WB, WB)],
                                 out.at[pl.ds(base_node + m * WB, WB)],
                                 wbsem)

        for j in range(WB_ITERS):
            m = j * NS + s

            @pl.when(m < WB_CHUNKS)
            def _():
                pltpu.make_async_copy(acc.at[pl.ds(m * WB, WB)],
                                      out.at[pl.ds(base_node + m * WB, WB)],
                                      wbsem).wait()
    else:
        def valid(j):
            return (j * NS + s) < WF_CHUNKS

        def f_loads(j, t):
            _, xb, yb = wsets[t]
            row0 = base_node + (j * NS + s) * WF
            pltpu.async_copy(e1.at[pl.ds(row0, WF)], xb, ldsems[t])
            pltpu.async_copy(e2.at[pl.ds(row0, WF)], yb, ldsems[t])

        def w_loads(j, t):
            _, xb, yb = wsets[t]
            row0 = base_node + (j * NS + s) * WF
            pltpu.make_async_copy(e1.at[pl.ds(row0, WF)], xb,
                                  ldsems[t]).wait()
            pltpu.make_async_copy(e2.at[pl.ds(row0, WF)], yb,
                                  ldsems[t]).wait()

        def f_store(j, t):
            ab = wsets[t][0]
            row0 = base_node + (j * NS + s) * WF
            pltpu.async_copy(ab, out.at[pl.ds(row0, WF)], stsems[t])

        def w_store(j, t):
            ab = wsets[t][0]
            row0 = base_node + (j * NS + s) * WF
            pltpu.make_async_copy(ab, out.at[pl.ds(row0, WF)],
                                  stsems[t]).wait()

        @pl.when(valid(0))
        def _():
            f_loads(0, 0)

        for j in range(WF_ITERS):
            t = j & 1
            if j >= 1:
                @pl.when(valid(j - 1))
                def _():
                    w_store(j - 1, 1 - t)

            if j + 1 < WF_ITERS:
                @pl.when(valid(j + 1))
                def _():
                    f_loads(j + 1, 1 - t)

            @pl.when(valid(j))
            def _():
                ab, xb, yb = wsets[t]
                pltpu.sync_copy(acc.at[pl.ds((j * NS + s) * WF, WF)], ab)
                w_loads(j, t)

                def mean_body(i, carry):
                    for k in range(2):
                        sl = pl.ds(k * 16, 16)
                        ab[i, sl] = (ab[i, sl] + xb[i, sl]
                                     + yb[i, sl]) * (1.0 / 3.0)
                    return carry

                lax.fori_loop(0, WF, mean_body, None)
                f_store(j, t)

        @pl.when(valid(WF_ITERS - 1))
        def _():
            w_store(WF_ITERS - 1, (WF_ITERS - 1) & 1)


def _layer_body(ego, psrc, pdst, pw, counts, out, acc, sbuf0, sbuf1, sbuf2,
                sbuf3, didx0, didx1, didx2, didx3, wbuf0, wbuf1, wbuf2,
                wbuf3, rows0, rows1, rows2, rows3, zbuf, isem0, isem1,
                isem2, isem3, gsem0, gsem1, gsem2, gsem3, ssem0, ssem1,
                ssem2, ssem3, wbsem):
    _layer_common(ego, psrc, pdst, pw, counts, out, None, None, acc,
                  [sbuf0, sbuf1, sbuf2, sbuf3],
                  [didx0, didx1, didx2, didx3],
                  [wbuf0, wbuf1, wbuf2, wbuf3],
                  [rows0, rows1, rows2, rows3], zbuf, None,
                  [isem0, isem1, isem2, isem3],
                  [gsem0, gsem1, gsem2, gsem3],
                  [ssem0, ssem1, ssem2, ssem3], wbsem, None, None,
                  final=False)


def _layer_final_body(ego, psrc, pdst, pw, counts, e1, e2, out, acc, sbuf0,
                      sbuf1, sbuf2, sbuf3, didx0, didx1, didx2, didx3,
                      wbuf0, wbuf1, wbuf2, wbuf3, rows0, rows1, rows2,
                      rows3, zbuf, ab0, xb0, yb0, ab1, xb1, yb1,
                      isem0, isem1, isem2, isem3,
                      gsem0, gsem1, gsem2, gsem3, ssem0, ssem1, ssem2,
                      ssem3, wbsem):
    _layer_common(ego, psrc, pdst, pw, counts, out, e1, e2, acc,
                  [sbuf0, sbuf1, sbuf2, sbuf3],
                  [didx0, didx1, didx2, didx3],
                  [wbuf0, wbuf1, wbuf2, wbuf3],
                  [rows0, rows1, rows2, rows3], zbuf,
                  [(ab0, xb0, yb0), (ab1, xb1, yb1)],
                  [isem0, isem1, isem2, isem3],
                  [gsem0, gsem1, gsem2, gsem3],
                  [ssem0, ssem1, ssem2, ssem3], wbsem,
                  [isem0, isem1], [isem2, isem3], final=True)


_ring_scratch = (
    [pltpu.VMEM((UE,), jnp.int32) for _ in range(NBUF)]
    + [pltpu.VMEM((1, UE), jnp.int32) for _ in range(NBUF)]
    + [pltpu.VMEM((UE,), jnp.float32) for _ in range(NBUF)]
    + [pltpu.VMEM((UE, EMB), jnp.float32) for _ in range(NBUF)]
)

_sem_scratch = [pltpu.SemaphoreType.DMA] * (3 * NBUF + 1)

_layer = pl.kernel(
    _layer_body,
    out_type=jax.ShapeDtypeStruct((NODES, EMB), jnp.float32),
    mesh=_mesh,
    compiler_params=pltpu.CompilerParams(use_tc_tiling_on_sc=False),
    scratch_types=(
        [pltpu.VMEM_SHARED((HALF, EMB), jnp.float32)]
        + _ring_scratch
        + [pltpu.VMEM((WB, EMB), jnp.float32)]
        + _sem_scratch
    ),
)

_layer_final = pl.kernel(
    _layer_final_body,
    out_type=jax.ShapeDtypeStruct((NODES, EMB), jnp.float32),
    mesh=_mesh,
    compiler_params=pltpu.CompilerParams(use_tc_tiling_on_sc=False),
    scratch_types=(
        [pltpu.VMEM_SHARED((HALF, EMB), jnp.float32)]
        + _ring_scratch
        + [pltpu.VMEM((WF, EMB), jnp.float32) for _ in range(7)]
        + _sem_scratch
    ),
)


def kernel(user_emb, item_emb, edge_weight, edge_index):
    ego = jnp.concatenate([user_emb, item_emb], axis=0)
    src = edge_index[0]
    dst = edge_index[1]
    pad = EP - E
    srcr = jnp.pad(src, (0, pad)).reshape(RP, 128)
    dstr = jnp.pad(dst, (0, pad)).reshape(RP, 128)
    wr = jnp.pad(edge_weight, (0, pad)).reshape(RP, 128)
    psrc, pdst, pw, counts = _partition(srcr, dstr, wr)
    e1 = _layer(ego, psrc, pdst, pw, counts)
    e2 = _layer(e1, psrc, pdst, pw, counts)
    fin = _layer_final(e2, psrc, pdst, pw, counts, e1, e2)
    return fin[:USER_N], fin[USER_N:]


# speedup vs baseline: 19.1316x; 1.0028x over previous
import functools

import jax
import jax.numpy as jnp
from jax import lax
from jax.experimental import pallas as pl
from jax.experimental.pallas import tpu as pltpu
from jax.experimental.pallas import tpu_sc as plsc

USER_N = 60000
ITEM_N = 40000
NODES = USER_N + ITEM_N
HALF = NODES // 2
EMB = 32
E = 1600000
NC = 2
NS = 16

SCAN_ROWS = 8
SCAN_E = SCAN_ROWS * 128
SPT = 98
EP = NS * SPT * SCAN_E
RP = EP // 128
REG = SPT * 1024 + 1024

UE = 128
NBUF = 4

WB = 80
WB_CHUNKS = HALF // WB
WB_ITERS = -(-WB_CHUNKS // NS)

WF = 40
WF_CHUNKS = HALF // WF
WF_ITERS = -(-WF_CHUNKS // NS)

_mesh = plsc.VectorSubcoreMesh(core_axis_name="c", subcore_axis_name="s")


FRING = 4096


def _partition_body(srcr, dstr, wr, psrc, pdst, pw, counts,
                    src_b0, dst_b0, w_b0, src_b1, dst_b1, w_b1,
                    f_src, f_dst, f_w, cbuf,
                    in_sem0, in_sem1, fl_sem0, fl_sem1):
    c = lax.axis_index("c")
    s = lax.axis_index("s")
    base_node = c * HALF
    tile_idx = c * NS + s
    base_el = tile_idx * REG

    src_b = [src_b0, src_b1]
    dst_b = [dst_b0, dst_b1]
    w_b = [w_b0, w_b1]
    in_sems = [in_sem0, in_sem1]
    fl_sems = [fl_sem0, fl_sem1]

    def fire_in(g, slot):
        r0 = (s * SPT + g) * SCAN_ROWS
        pltpu.async_copy(srcr.at[pl.ds(r0, SCAN_ROWS)], src_b[slot],
                         in_sems[slot])
        pltpu.async_copy(dstr.at[pl.ds(r0, SCAN_ROWS)], dst_b[slot],
                         in_sems[slot])
        pltpu.async_copy(wr.at[pl.ds(r0, SCAN_ROWS)], w_b[slot],
                         in_sems[slot])

    def wait_in(g, slot):
        r0 = (s * SPT + g) * SCAN_ROWS
        pltpu.make_async_copy(srcr.at[pl.ds(r0, SCAN_ROWS)], src_b[slot],
                              in_sems[slot]).wait()
        pltpu.make_async_copy(dstr.at[pl.ds(r0, SCAN_ROWS)], dst_b[slot],
                              in_sems[slot]).wait()
        pltpu.make_async_copy(wr.at[pl.ds(r0, SCAN_ROWS)], w_b[slot],
                              in_sems[slot]).wait()

    def wait_flush(sem):
        pltpu.make_async_copy(f_src.at[pl.ds(0, 1024)],
                              psrc.at[pl.ds(base_el, 1024)], sem).wait()
        pltpu.make_async_copy(f_dst.at[pl.ds(0, 1024)],
                              pdst.at[pl.ds(base_el, 1024)], sem).wait()
        pltpu.make_async_copy(f_w.at[pl.ds(0, 1024)],
                              pw.at[pl.ds(base_el, 1024)], sem).wait()

    def fire_flush(fq, sem):
        q0 = (fq & 3) * 1024
        dst_el = base_el + fq * 1024
        pltpu.async_copy(f_src.at[pl.ds(q0, 1024)],
                         psrc.at[pl.ds(dst_el, 1024)], sem)
        pltpu.async_copy(f_dst.at[pl.ds(q0, 1024)],
                         pdst.at[pl.ds(dst_el, 1024)], sem)
        pltpu.async_copy(f_w.at[pl.ds(q0, 1024)],
                         pw.at[pl.ds(dst_el, 1024)], sem)

    fire_in(0, 0)
    fire_in(1, 1)

    def sub_chunk(g, slot, carry):
        tot, fq = carry
        wait_in(g, slot)
        for i in range(SCAN_ROWS):
            for k in range(8):
                sl = pl.ds(k * 16, 16)
                d = dst_b[slot][i, sl] - base_node
                m = (d >= 0) & (d < HALF)
                pos = plsc.cumsum(m.astype(jnp.int32))
                idx = (pos + (tot - 1)) & (FRING - 1)
                plsc.store_scatter(f_src, [idx], src_b[slot][i, sl], mask=m)
                plsc.store_scatter(f_dst, [idx], d, mask=m)
                plsc.store_scatter(f_w, [idx], w_b[slot][i, sl], mask=m)
                tot = tot + pos[15]

        @pl.when(g + 2 < SPT)
        def _():
            fire_in(g + 2, slot)

        unflushed = tot - fq * 1024
        need = unflushed >= 1024
        even = (fq & 1) == 0

        @pl.when(need & even & (fq >= 2))
        def _():
            wait_flush(fl_sem0)

        @pl.when(need & even)
        def _():
            fire_flush(fq, fl_sem0)

        @pl.when(need & (~even) & (fq >= 2))
        def _():
            wait_flush(fl_sem1)

        @pl.when(need & (~even))
        def _():
            fire_flush(fq, fl_sem1)

        return tot, fq + (unflushed >> 10)

    def super_chunk(k, carry):
        carry = sub_chunk(2 * k, 0, carry)
        carry = sub_chunk(2 * k + 1, 1, carry)
        return carry

    tot, fq = lax.fori_loop(0, SPT // 2, super_chunk,
                            (jnp.int32(0), jnp.int32(0)))

    evenq = (fq & 1) == 0

    @pl.when((fq >= 1) & evenq)
    def _():
        wait_flush(fl_sem1)

    @pl.when((fq >= 1) & (~evenq))
    def _():
        wait_flush(fl_sem0)

    @pl.when((fq >= 2) & evenq)
    def _():
        wait_flush(fl_sem0)

    @pl.when((fq >= 2) & (~evenq))
    def _():
        wait_flush(fl_sem1)

    lanes = lax.iota(jnp.int32, 16)
    mall = lanes >= 0
    zi = jnp.zeros((16,), jnp.int32)
    zf = jnp.zeros((16,), jnp.float32)
    for j in range(8):
        idxv = (lanes + (tot + j * 16)) & (FRING - 1)
        plsc.store_scatter(f_src, [idxv], zi, mask=mall)
        plsc.store_scatter(f_dst, [idxv], zi, mask=mall)
        plsc.store_scatter(f_w, [idxv], zf, mask=mall)

    q0 = (fq & 3) * 1024
    dst_el = base_el + fq * 1024
    pltpu.sync_copy(f_src.at[pl.ds(q0, 1024)], psrc.at[pl.ds(dst_el, 1024)])
    pltpu.sync_copy(f_dst.at[pl.ds(q0, 1024)], pdst.at[pl.ds(dst_el, 1024)])
    pltpu.sync_copy(f_w.at[pl.ds(q0, 1024)], pw.at[pl.ds(dst_el, 1024)])

    rem = tot - fq * 1024
    cnt = fq * 8 + ((rem + UE - 1) >> 7)
    cbuf[...] = jnp.full((16,), cnt, jnp.int32)
    pltpu.sync_copy(cbuf, counts.at[pl.ds(tile_idx * 16, 16)])


_partition = pl.kernel(
    _partition_body,
    out_type=(
        jax.ShapeDtypeStruct((NC * NS * REG,), jnp.int32),
        jax.ShapeDtypeStruct((NC * NS * REG,), jnp.int32),
        jax.ShapeDtypeStruct((NC * NS * REG,), jnp.float32),
        jax.ShapeDtypeStruct((NC * NS * 16,), jnp.int32),
    ),
    mesh=_mesh,
    compiler_params=pltpu.CompilerParams(use_tc_tiling_on_sc=False,
                                         needs_layout_passes=False),
    scratch_types=(
        [pltpu.VMEM((SCAN_ROWS, 128), jnp.int32),
         pltpu.VMEM((SCAN_ROWS, 128), jnp.int32),
         pltpu.VMEM((SCAN_ROWS, 128), jnp.float32)] * 2
        + [pltpu.VMEM((FRING,), jnp.int32),
           pltpu.VMEM((FRING,), jnp.int32),
           pltpu.VMEM((FRING,), jnp.float32),
           pltpu.VMEM((16,), jnp.int32)]
        + [pltpu.SemaphoreType.DMA] * 4
    ),
)


def _layer_common(ego, psrc, pdst, pw, counts, out, e1, e2, acc,
                  sbuf, didx, wbuf, rows, zbuf, wsets,
                  isems, gsems, ssems, wbsem, ldsems, stsems, final):
    c = lax.axis_index("c")
    s = lax.axis_index("s")
    base_node = c * HALF
    tile_idx = c * NS + s
    base_el = tile_idx * REG

    zw = WF if final else WB
    zn = WF_ITERS if final else WB_ITERS
    zc = WF_CHUNKS if final else WB_CHUNKS

    def z_body(i, carry):
        zbuf[i, pl.ds(0, 16)] = jnp.zeros((16,), jnp.float32)
        zbuf[i, pl.ds(16, 16)] = jnp.zeros((16,), jnp.float32)
        return carry

    lax.fori_loop(0, zw, z_body, None)
    pltpu.sync_copy(counts.at[pl.ds(tile_idx * 16, 16)],
                    sbuf[0].at[pl.ds(0, 16)])
    n_units = sbuf[0][pl.ds(0, 16)][0]
    for j in range(zn):
        m = j * NS + s

        @pl.when(m < zc)
        def _():
            pltpu.async_copy(zbuf, acc.at[pl.ds(m * zw, zw)], wbsem)

    def fire_idx(u, slot):
        el = base_el + u * UE
        pltpu.async_copy(psrc.at[pl.ds(el, UE)], sbuf[slot], isems[slot])
        pltpu.async_copy(pdst.at[pl.ds(el, UE)], didx[slot].at[0],
                         isems[slot])
        pltpu.async_copy(pw.at[pl.ds(el, UE)], wbuf[slot], isems[slot])

    def wait_idx(u, slot):
        el = base_el + u * UE
        pltpu.make_async_copy(psrc.at[pl.ds(el, UE)], sbuf[slot],
                              isems[slot]).wait()
        pltpu.make_async_copy(pdst.at[pl.ds(el, UE)], didx[slot].at[0],
                              isems[slot]).wait()
        pltpu.make_async_copy(pw.at[pl.ds(el, UE)], wbuf[slot],
                              isems[slot]).wait()

    def fire_gather(slot):
        pltpu.async_copy(ego.at[sbuf[slot]], rows[slot], gsems[slot])

    def wait_gather(slot):
        pltpu.make_async_copy(ego.at[sbuf[slot]], rows[slot],
                              gsems[slot]).wait()

    def fire_scatter(slot):
        pltpu.async_copy(rows[slot], acc.at[didx[slot].at[0]], ssems[slot],
                         add=True)

    def wait_scatter(slot):
        pltpu.make_async_copy(rows[slot], acc.at[didx[slot].at[0]],
                              ssems[slot]).wait()

    @pl.when(n_units > 0)
    def _():
        fire_idx(0, 0)

    @pl.when(n_units > 1)
    def _():
        fire_idx(1, 1)

    @pl.when(n_units > 0)
    def _():
        wait_idx(0, 0)
        fire_gather(0)

    for j in range(zn):
        m = j * NS + s

        @pl.when(m < zc)
        def _():
            pltpu.make_async_copy(zbuf, acc.at[pl.ds(m * zw, zw)],
                                  wbsem).wait()

    plsc.subcore_barrier()

    def super_body(si, carry):
        t0 = si * NBUF
        for j in range(NBUF):
            t = t0 + j
            sl = j
            sl1 = (j + 1) % NBUF
            sl2 = (j + 2) % NBUF

            @pl.when(t < n_units)
            def _():
                @pl.when(t + 1 < n_units)
                def _():
                    wait_idx(t + 1, sl1)
                    fire_gather(sl1)

                @pl.when(t + 2 < n_units)
                def _():
                    @pl.when(t >= 2)
                    def _():
                        wait_scatter(sl2)
                    fire_idx(t + 2, sl2)

                wait_gather(sl)

                for k in range(UE // 16):
                    wv = wbuf[sl][pl.ds(k * 16, 16)]
                    for l in range(16):
                        e = k * 16 + l
                        wsc = wv[l]
                        rows[sl][e, pl.ds(0, 16)] = (
                            rows[sl][e, pl.ds(0, 16)] * wsc)
                        rows[sl][e, pl.ds(16, 16)] = (
                            rows[sl][e, pl.ds(16, 16)] * wsc)
                fire_scatter(sl)

        return carry

    lax.fori_loop(0, (n_units + NBUF - 1) // NBUF, super_body, None)

    for j in range(NBUF):
        @pl.when(j < n_units)
        def _():
            pltpu.make_async_copy(rows[j], acc.at[didx[j].at[0]],
                                  ssems[j]).wait()

    plsc.subcore_barrier()

    if not final:
        for j in range(WB_ITERS):
            m = j * NS + s

            @pl.when(m < WB_CHUNKS)
            def _():
                pltpu.async_copy(acc.at[pl.ds(m * WB, WB)],
                                 out.at[pl.ds(base_node + m * WB, WB)],
                                 wbsem)

        for j in range(WB_ITERS):
            m = j * NS + s

            @pl.when(m < WB_CHUNKS)
            def _():
                pltpu.make_async_copy(acc.at[pl.ds(m * WB, WB)],
                                      out.at[pl.ds(base_node + m * WB, WB)],
                                      wbsem).wait()
    else:
        def valid(j):
            return (j * NS + s) < WF_CHUNKS

        def f_loads(j, t):
            _, xb, yb = wsets[t]
            row0 = base_node + (j * NS + s) * WF
            pltpu.async_copy(e1.at[pl.ds(row0, WF)], xb, ldsems[t])
            pltpu.async_copy(e2.at[pl.ds(row0, WF)], yb, ldsems[t])

        def w_loads(j, t):
            _, xb, yb = wsets[t]
            row0 = base_node + (j * NS + s) * WF
            pltpu.make_async_copy(e1.at[pl.ds(row0, WF)], xb,
                                  ldsems[t]).wait()
            pltpu.make_async_copy(e2.at[pl.ds(row0, WF)], yb,
                                  ldsems[t]).wait()

        def f_store(j, t):
            ab = wsets[t][0]
            row0 = base_node + (j * NS + s) * WF
            pltpu.async_copy(ab, out.at[pl.ds(row0, WF)], stsems[t])

        def w_store(j, t):
            ab = wsets[t][0]
            row0 = base_node + (j * NS + s) * WF
            pltpu.make_async_copy(ab, out.at[pl.ds(row0, WF)],
                                  stsems[t]).wait()

        @pl.when(valid(0))
        def _():
            f_loads(0, 0)

        for j in range(WF_ITERS):
            t = j & 1
            if j >= 1:
                @pl.when(valid(j - 1))
                def _():
                    w_store(j - 1, 1 - t)

            if j + 1 < WF_ITERS:
                @pl.when(valid(j + 1))
                def _():
                    f_loads(j + 1, 1 - t)

            @pl.when(valid(j))
            def _():
                ab, xb, yb = wsets[t]
                pltpu.sync_copy(acc.at[pl.ds((j * NS + s) * WF, WF)], ab)
                w_loads(j, t)

                def mean_body(i, carry):
                    for k in range(2):
                        sl = pl.ds(k * 16, 16)
                        ab[i, sl] = (ab[i, sl] + xb[i, sl]
                                     + yb[i, sl]) * (1.0 / 3.0)
                    return carry

                lax.fori_loop(0, WF, mean_body, None)
                f_store(j, t)

        @pl.when(valid(WF_ITERS - 1))
        def _():
            w_store(WF_ITERS - 1, (WF_ITERS - 1) & 1)


def _layer_body(ego, psrc, pdst, pw, counts, out, acc, sbuf0, sbuf1, sbuf2,
                sbuf3, didx0, didx1, didx2, didx3, wbuf0, wbuf1, wbuf2,
                wbuf3, rows0, rows1, rows2, rows3, zbuf, isem0, isem1,
                isem2, isem3, gsem0, gsem1, gsem2, gsem3, ssem0, ssem1,
                ssem2, ssem3, wbsem):
    _layer_common(ego, psrc, pdst, pw, counts, out, None, None, acc,
                  [sbuf0, sbuf1, sbuf2, sbuf3],
                  [didx0, didx1, didx2, didx3],
                  [wbuf0, wbuf1, wbuf2, wbuf3],
                  [rows0, rows1, rows2, rows3], zbuf, None,
                  [isem0, isem1, isem2, isem3],
                  [gsem0, gsem1, gsem2, gsem3],
                  [ssem0, ssem1, ssem2, ssem3], wbsem, None, None,
                  final=False)


def _layer_final_body(ego, psrc, pdst, pw, counts, e1, e2, out, acc, sbuf0,
                      sbuf1, sbuf2, sbuf3, didx0, didx1, didx2, didx3,
                      wbuf0, wbuf1, wbuf2, wbuf3, rows0, rows1, rows2,
                      rows3, zbuf, ab0, xb0, yb0, ab1, xb1, yb1,
                      isem0, isem1, isem2, isem3,
                      gsem0, gsem1, gsem2, gsem3, ssem0, ssem1, ssem2,
                      ssem3, wbsem):
    _layer_common(ego, psrc, pdst, pw, counts, out, e1, e2, acc,
                  [sbuf0, sbuf1, sbuf2, sbuf3],
                  [didx0, didx1, didx2, didx3],
                  [wbuf0, wbuf1, wbuf2, wbuf3],
                  [rows0, rows1, rows2, rows3], zbuf,
                  [(ab0, xb0, yb0), (ab1, xb1, yb1)],
                  [isem0, isem1, isem2, isem3],
                  [gsem0, gsem1, gsem2, gsem3],
                  [ssem0, ssem1, ssem2, ssem3], wbsem,
                  [isem0, isem1], [isem2, isem3], final=True)


_ring_scratch = (
    [pltpu.VMEM((UE,), jnp.int32) for _ in range(NBUF)]
    + [pltpu.VMEM((1, UE), jnp.int32) for _ in range(NBUF)]
    + [pltpu.VMEM((UE,), jnp.float32) for _ in range(NBUF)]
    + [pltpu.VMEM((UE, EMB), jnp.float32) for _ in range(NBUF)]
)

_sem_scratch = [pltpu.SemaphoreType.DMA] * (3 * NBUF + 1)

_layer = pl.kernel(
    _layer_body,
    out_type=jax.ShapeDtypeStruct((NODES, EMB), jnp.float32),
    mesh=_mesh,
    compiler_params=pltpu.CompilerParams(use_tc_tiling_on_sc=False),
    scratch_types=(
        [pltpu.VMEM_SHARED((HALF, EMB), jnp.float32)]
        + _ring_scratch
        + [pltpu.VMEM((WB, EMB), jnp.float32)]
        + _sem_scratch
    ),
)

_layer_final = pl.kernel(
    _layer_final_body,
    out_type=jax.ShapeDtypeStruct((NODES, EMB), jnp.float32),
    mesh=_mesh,
    compiler_params=pltpu.CompilerParams(use_tc_tiling_on_sc=False),
    scratch_types=(
        [pltpu.VMEM_SHARED((HALF, EMB), jnp.float32)]
        + _ring_scratch
        + [pltpu.VMEM((WF, EMB), jnp.float32) for _ in range(7)]
        + _sem_scratch
    ),
)


def kernel(user_emb, item_emb, edge_weight, edge_index):
    ego = jnp.concatenate([user_emb, item_emb], axis=0)
    src = edge_index[0]
    dst = edge_index[1]
    pad = EP - E
    srcr = jnp.pad(src, (0, pad)).reshape(RP, 128)
    dstr = jnp.pad(dst, (0, pad)).reshape(RP, 128)
    wr = jnp.pad(edge_weight, (0, pad)).reshape(RP, 128)
    psrc, pdst, pw, counts = _partition(srcr, dstr, wr)
    e1 = _layer(ego, psrc, pdst, pw, counts)
    e2 = _layer(e1, psrc, pdst, pw, counts)
    fin = _layer_final(e2, psrc, pdst, pw, counts, e1, e2)
    return fin[:USER_N], fin[USER_N:]
